# baseline (speedup 1.0000x reference)
import functools

import jax
import jax.numpy as jnp
from jax import lax
from jax.experimental import pallas as pl
from jax.experimental.pallas import tpu as pltpu
from jax.experimental.pallas import tpu_sc as plsc

N = 10000
E = 320000
D = 128
H = 128
BN_INV = 1.0 / (1.0 + 1e-5) ** 0.5

NC = 2
NS = 16
NW = NC * NS
EP = E // NW
CH = 80
NCHUNK = EP // CH
NP = 624
NTAIL = N - NP * NS


def _zero_rows(rows):
    z = jnp.zeros((16,), jnp.float32)

    def body(r, c):
        for j in range(D // 16):
            rows[r, pl.ds(j * 16, 16)] = z
        return c

    lax.fori_loop(0, CH, body, 0)


_SC_MESH = plsc.VectorSubcoreMesh(core_axis_name="c", subcore_axis_name="s",
                                  num_cores=NC, num_subcores=NS)
_SC_PARAMS = pltpu.CompilerParams(needs_layout_passes=False)


@functools.partial(
    pl.kernel,
    out_type=jax.ShapeDtypeStruct((2 * N, D), jnp.float32),
    mesh=_SC_MESH,
    compiler_params=_SC_PARAMS,
    scratch_types=[
        pltpu.VMEM((EP,), jnp.int32),
        pltpu.VMEM((CH,), jnp.int32),
        pltpu.VMEM((CH,), jnp.int32),
        pltpu.VMEM((CH, D), jnp.float32),
        pltpu.VMEM((CH, D), jnp.float32),
        pltpu.VMEM_SHARED((N, D), jnp.float32),
        pltpu.SemaphoreType.DMA,
        pltpu.SemaphoreType.DMA,
        pltpu.SemaphoreType.DMA,
        pltpu.SemaphoreType.DMA,
        pltpu.SemaphoreType.DMA,
    ],
)
def _sc_segsum(x_hbm, src_hbm, dst_hbm, out_hbm,
               sidx, didx0, didx1, rows0, rows1, acc,
               sem0, sem1, semd0, semd1, semi):
    cid = lax.axis_index("c")
    sid = lax.axis_index("s")
    wid = cid * NS + sid
    ebase = wid * EP

    pltpu.make_async_copy(src_hbm.at[pl.ds(ebase, EP)], sidx, semi).start()

    _zero_rows(rows0)
    nbase = sid * NP
    full = NP // CH
    for c in range(full):
        pltpu.sync_copy(rows0, acc.at[pl.ds(nbase + c * CH, CH)])
    rem = NP - full * CH
    if rem:
        pltpu.sync_copy(rows0.at[pl.ds(0, rem)], acc.at[pl.ds(nbase + full * CH, rem)])

    @pl.when(sid == NS - 1)
    def _tail_zero():
        pltpu.sync_copy(rows0.at[pl.ds(0, NTAIL)], acc.at[pl.ds(NP * NS, NTAIL)])

    pltpu.make_async_copy(src_hbm.at[pl.ds(ebase, EP)], sidx, semi).wait()
    plsc.subcore_barrier()

    def fire(i, buf, dbuf, sem, semd):
        pltpu.make_async_copy(dst_hbm.at[pl.ds(ebase + i * CH, CH)], dbuf,
                              semd).start()
        pltpu.make_async_copy(x_hbm.at[sidx.at[pl.ds(i * CH, CH)]], buf,
                              sem).start()

    def drain(buf, dbuf, sem, semd):
        pltpu.make_async_copy(x_hbm.at[sidx.at[pl.ds(0, CH)]], buf, sem).wait()
        pltpu.make_async_copy(dst_hbm.at[pl.ds(ebase, CH)], dbuf, semd).wait()

    fire(0, rows0, didx0, sem0, semd0)

    def pair(g, c):
        i0 = 2 * g
        fire(i0 + 1, rows1, didx1, sem1, semd1)
        drain(rows0, didx0, sem0, semd0)
        pltpu.sync_copy(rows0, acc.at[didx0], add=True)
        fire(i0 + 2, rows0, didx0, sem0, semd0)
        drain(rows1, didx1, sem1, semd1)
        pltpu.sync_copy(rows1, acc.at[didx1], add=True)
        return c

    lax.fori_loop(0, (NCHUNK - 1) // 2, pair, 0)
    drain(rows0, didx0, sem0, semd0)
    pltpu.sync_copy(rows0, acc.at[didx0], add=True)
    plsc.subcore_barrier()

    pltpu.sync_copy(acc.at[pl.ds(nbase, NP)],
                    out_hbm.at[pl.ds(cid * N + nbase, NP)])

    @pl.when(sid == NS - 1)
    def _tail_dump():
        pltpu.sync_copy(acc.at[pl.ds(NP * NS, NTAIL)],
                        out_hbm.at[pl.ds(cid * N + NP * NS, NTAIL)])


HCH = 128
HNCH = EP // HCH
HTAIL = EP - HNCH * HCH


@functools.partial(
    pl.kernel,
    out_type=jax.ShapeDtypeStruct((E,), jnp.float32),
    mesh=_SC_MESH,
    compiler_params=_SC_PARAMS,
    scratch_types=[
        pltpu.VMEM((EP,), jnp.int32),
        pltpu.VMEM((EP,), jnp.int32),
        pltpu.VMEM((HCH, D), jnp.float32),
        pltpu.VMEM((HCH, D), jnp.float32),
        pltpu.VMEM((HCH, D), jnp.float32),
        pltpu.VMEM((HCH, D), jnp.float32),
        pltpu.VMEM((D,), jnp.float32),
        pltpu.VMEM((EP,), jnp.float32),
        pltpu.VMEM((16, 16), jnp.float32),
        pltpu.SemaphoreType.DMA,
        pltpu.SemaphoreType.DMA,
        pltpu.SemaphoreType.DMA,
    ],
)
def _sc_head(a_hbm, b_hbm, w2_hbm, src_hbm, dst_hbm, s_hbm,
             sidx, didx, a0, a1, b0, b1, w2v, sbuf, ptile, sem0, sem1, semi):
    cid = lax.axis_index("c")
    sid = lax.axis_index("s")
    wid = cid * NS + sid
    ebase = wid * EP

    pltpu.make_async_copy(src_hbm.at[pl.ds(ebase, EP)], sidx, semi).start()
    pltpu.make_async_copy(dst_hbm.at[pl.ds(ebase, EP)], didx, semi).start()
    pltpu.sync_copy(w2_hbm, w2v)
    w2s = [w2v[pl.ds(j * 16, 16)] for j in range(D // 16)]
    lane = lax.iota(jnp.int32, 16)
    pltpu.make_async_copy(src_hbm.at[pl.ds(ebase, EP)], sidx, semi).wait()
    pltpu.make_async_copy(dst_hbm.at[pl.ds(ebase, EP)], didx, semi).wait()

    def fire(i, abuf, bbuf, sem):
        pltpu.make_async_copy(a_hbm.at[sidx.at[pl.ds(i * HCH, HCH)]], abuf,
                              sem).start()
        pltpu.make_async_copy(b_hbm.at[didx.at[pl.ds(i * HCH, HCH)]], bbuf,
                              sem).start()

    def drain(abuf, bbuf, sem):
        pltpu.make_async_copy(a_hbm.at[sidx.at[pl.ds(0, HCH)]], abuf, sem).wait()
        pltpu.make_async_copy(b_hbm.at[didx.at[pl.ds(0, HCH)]], bbuf, sem).wait()

    def compute(base, arows, brows, ngroups):
        def grp(g, cc):
            for rr in range(16):
                r = g * 16 + rr
                dot = jnp.zeros((16,), jnp.float32)
                for j in range(D // 16):
                    t = arows[r, pl.ds(j * 16, 16)] + brows[r, pl.ds(j * 16, 16)]
                    dot = dot + jnp.maximum(t, 0.0) * w2s[j]
                ptile[rr, pl.ds(0, 16)] = dot
            svec = jnp.zeros((16,), jnp.float32)
            for k in range(16):
                svec = svec + plsc.load_gather(
                    ptile, [lane, jnp.full((16,), k, jnp.int32)])
            sbuf[pl.ds(base + g * 16, 16)] = svec
            return cc

        lax.fori_loop(0, ngroups, grp, 0)

    fire(0, a0, b0, sem0)

    def pair(g, c):
        i0 = 2 * g
        fire(i0 + 1, a1, b1, sem1)
        drain(a0, b0, sem0)
        compute(i0 * HCH, a0, b0, HCH // 16)
        fire(i0 + 2, a0, b0, sem0)
        drain(a1, b1, sem1)
        compute((i0 + 1) * HCH, a1, b1, HCH // 16)
        return c

    lax.fori_loop(0, (HNCH - 2) // 2, pair, 0)
    fire(HNCH - 1, a1, b1, sem1)
    drain(a0, b0, sem0)
    compute((HNCH - 2) * HCH, a0, b0, HCH // 16)
    pltpu.make_async_copy(a_hbm.at[sidx.at[pl.ds(HNCH * HCH, HTAIL)]],
                          a0.at[pl.ds(0, HTAIL)], sem0).start()
    pltpu.make_async_copy(b_hbm.at[didx.at[pl.ds(HNCH * HCH, HTAIL)]],
                          b0.at[pl.ds(0, HTAIL)], sem0).start()
    drain(a1, b1, sem1)
    compute((HNCH - 1) * HCH, a1, b1, HCH // 16)
    pltpu.make_async_copy(a_hbm.at[sidx.at[pl.ds(0, HTAIL)]],
                          a0.at[pl.ds(0, HTAIL)], sem0).wait()
    pltpu.make_async_copy(b_hbm.at[didx.at[pl.ds(0, HTAIL)]],
                          b0.at[pl.ds(0, HTAIL)], sem0).wait()
    compute(HNCH * HCH, a0, b0, HTAIL // 16)
    pltpu.sync_copy(sbuf, s_hbm.at[pl.ds(wid * EP, EP)])


NB = 400
GRID = N // NB


def _mlp_body(x_ref, a0_ref, a1_ref, w1_ref, b1_ref, g_ref, be_ref,
              w2_ref, b2_ref, o_ref):
    h = x_ref[...] + a0_ref[...] + a1_ref[...]
    t = jnp.dot(h, w1_ref[...], preferred_element_type=jnp.float32) + b1_ref[...]
    t = t * (g_ref[...] * BN_INV) + be_ref[...]
    t = jnp.maximum(t, 0.0)
    t = jnp.dot(t, w2_ref[...], preferred_element_type=jnp.float32) + b2_ref[...]
    o_ref[...] = jnp.maximum(t, 0.0)


def _mlp2_body(x_ref, a0_ref, a1_ref, w1_ref, b1_ref, g_ref, be_ref,
               w2_ref, b2_ref, mw1_ref, mb1_ref, a_out, b_out):
    h = x_ref[...] + a0_ref[...] + a1_ref[...]
    t = jnp.dot(h, w1_ref[...], preferred_element_type=jnp.float32) + b1_ref[...]
    t = t * (g_ref[...] * BN_INV) + be_ref[...]
    t = jnp.maximum(t, 0.0)
    t = jnp.dot(t, w2_ref[...], preferred_element_type=jnp.float32) + b2_ref[...]
    h2 = jnp.maximum(t, 0.0)
    a_out[...] = jnp.dot(h2, mw1_ref[0:H, :],
                         preferred_element_type=jnp.float32) + mb1_ref[...]
    b_out[...] = jnp.dot(h2, mw1_ref[H:2 * H, :],
                         preferred_element_type=jnp.float32)


_row_spec = pl.BlockSpec((NB, D), lambda i: (i, 0))
_row2_spec = pl.BlockSpec((NB, D), lambda i: (i + GRID, 0))
_w_spec = pl.BlockSpec((D, D), lambda i: (0, 0))
_b_spec = pl.BlockSpec((1, D), lambda i: (0, 0))

_mlp = pl.pallas_call(
    _mlp_body,
    grid=(GRID,),
    in_specs=[_row_spec, _row_spec, _row2_spec,
              _w_spec, _b_spec, _b_spec, _b_spec, _w_spec, _b_spec],
    out_specs=_row_spec,
    out_shape=jax.ShapeDtypeStruct((N, D), jnp.float32),
)

_mlp2 = pl.pallas_call(
    _mlp2_body,
    grid=(GRID,),
    in_specs=[_row_spec, _row_spec, _row2_spec,
              _w_spec, _b_spec, _b_spec, _b_spec, _w_spec, _b_spec,
              pl.BlockSpec((2 * H, D), lambda i: (0, 0)), _b_spec],
    out_specs=[_row_spec, _row_spec],
    out_shape=[jax.ShapeDtypeStruct((N, D), jnp.float32),
               jax.ShapeDtypeStruct((N, D), jnp.float32)],
)


def _softmax_body(s_ref, o_ref):
    s = s_ref[...]
    m = jnp.max(s)
    e = jnp.exp(s - m)
    o_ref[...] = e / jnp.sum(e)


_softmax = pl.pallas_call(
    _softmax_body,
    out_shape=jax.ShapeDtypeStruct((E // D, D), jnp.float32),
)


def kernel(x, edge_index, c0_w1, c0_b1, c0_g, c0_be, c0_w2, c0_b2,
           c1_w1, c1_b1, c1_g, c1_be, c1_w2, c1_b2,
           m_w1, m_b1, m_w2, m_b2):
    src = edge_index[0]
    dst = edge_index[1]
    r = lambda v: v.reshape(1, -1)

    agg0 = _sc_segsum(x, src, dst)
    h1 = _mlp(x, agg0, agg0, c0_w1, r(c0_b1), r(c0_g), r(c0_be),
              c0_w2, r(c0_b2))
    agg1 = _sc_segsum(h1, src, dst)
    a, b = _mlp2(h1, agg1, agg1, c1_w1, r(c1_b1), r(c1_g), r(c1_be),
                 c1_w2, r(c1_b2), m_w1, r(m_b1))
    s = _sc_head(a, b, m_w2.reshape(-1), src, dst)
    return _softmax(s.reshape(E // D, D)).reshape(-1)


# speedup vs baseline: 1.0313x; 1.0313x over previous
import functools

import jax
import jax.numpy as jnp
from jax import lax
from jax.experimental import pallas as pl
from jax.experimental.pallas import tpu as pltpu
from jax.experimental.pallas import tpu_sc as plsc

N = 10000
E = 320000
D = 128
H = 128
BN_INV = 1.0 / (1.0 + 1e-5) ** 0.5

NC = 2
NS = 16
NW = NC * NS
EP = E // NW
CH = 80
NCHUNK = EP // CH
NP = 624
NTAIL = N - NP * NS


def _zero_rows(rows):
    z = jnp.zeros((16,), jnp.float32)

    def body(r, c):
        for j in range(D // 16):
            rows[r, pl.ds(j * 16, 16)] = z
        return c

    lax.fori_loop(0, CH, body, 0)


_SC_MESH = plsc.VectorSubcoreMesh(core_axis_name="c", subcore_axis_name="s",
                                  num_cores=NC, num_subcores=NS)
_SC_PARAMS = pltpu.CompilerParams(needs_layout_passes=False)


@functools.partial(
    pl.kernel,
    out_type=jax.ShapeDtypeStruct((2 * N, D), jnp.float32),
    mesh=_SC_MESH,
    compiler_params=_SC_PARAMS,
    scratch_types=[
        pltpu.VMEM((EP,), jnp.int32),
        pltpu.VMEM((CH,), jnp.int32),
        pltpu.VMEM((CH,), jnp.int32),
        pltpu.VMEM((CH, D), jnp.float32),
        pltpu.VMEM((CH, D), jnp.float32),
        pltpu.VMEM_SHARED((N, D), jnp.float32),
        pltpu.SemaphoreType.DMA,
        pltpu.SemaphoreType.DMA,
        pltpu.SemaphoreType.DMA,
        pltpu.SemaphoreType.DMA,
        pltpu.SemaphoreType.DMA,
    ],
)
def _sc_segsum(x_hbm, src_hbm, dst_hbm, out_hbm,
               sidx, didx0, didx1, rows0, rows1, acc,
               sem0, sem1, semd0, semd1, semi):
    cid = lax.axis_index("c")
    sid = lax.axis_index("s")
    wid = cid * NS + sid
    ebase = wid * EP

    pltpu.make_async_copy(src_hbm.at[pl.ds(ebase, EP)], sidx, semi).start()

    _zero_rows(rows0)
    nbase = sid * NP
    full = NP // CH
    for c in range(full):
        pltpu.sync_copy(rows0, acc.at[pl.ds(nbase + c * CH, CH)])
    rem = NP - full * CH
    if rem:
        pltpu.sync_copy(rows0.at[pl.ds(0, rem)], acc.at[pl.ds(nbase + full * CH, rem)])

    @pl.when(sid == NS - 1)
    def _tail_zero():
        pltpu.sync_copy(rows0.at[pl.ds(0, NTAIL)], acc.at[pl.ds(NP * NS, NTAIL)])

    pltpu.make_async_copy(src_hbm.at[pl.ds(ebase, EP)], sidx, semi).wait()
    plsc.subcore_barrier()

    def fire(i, buf, dbuf, sem, semd):
        pltpu.make_async_copy(dst_hbm.at[pl.ds(ebase + i * CH, CH)], dbuf,
                              semd).start()
        pltpu.make_async_copy(x_hbm.at[sidx.at[pl.ds(i * CH, CH)]], buf,
                              sem).start()

    def drain(buf, dbuf, sem, semd):
        pltpu.make_async_copy(x_hbm.at[sidx.at[pl.ds(0, CH)]], buf, sem).wait()
        pltpu.make_async_copy(dst_hbm.at[pl.ds(ebase, CH)], dbuf, semd).wait()

    fire(0, rows0, didx0, sem0, semd0)

    def pair(g, c):
        i0 = 2 * g
        fire(i0 + 1, rows1, didx1, sem1, semd1)
        drain(rows0, didx0, sem0, semd0)
        pltpu.sync_copy(rows0, acc.at[didx0], add=True)
        fire(i0 + 2, rows0, didx0, sem0, semd0)
        drain(rows1, didx1, sem1, semd1)
        pltpu.sync_copy(rows1, acc.at[didx1], add=True)
        return c

    lax.fori_loop(0, (NCHUNK - 1) // 2, pair, 0)
    drain(rows0, didx0, sem0, semd0)
    pltpu.sync_copy(rows0, acc.at[didx0], add=True)
    plsc.subcore_barrier()

    pltpu.sync_copy(acc.at[pl.ds(nbase, NP)],
                    out_hbm.at[pl.ds(cid * N + nbase, NP)])

    @pl.when(sid == NS - 1)
    def _tail_dump():
        pltpu.sync_copy(acc.at[pl.ds(NP * NS, NTAIL)],
                        out_hbm.at[pl.ds(cid * N + NP * NS, NTAIL)])


HCH = 128
HNCH = EP // HCH
HTAIL = EP - HNCH * HCH


@functools.partial(
    pl.kernel,
    out_type=jax.ShapeDtypeStruct((E,), jnp.float32),
    mesh=_SC_MESH,
    compiler_params=_SC_PARAMS,
    scratch_types=[
        pltpu.VMEM((EP,), jnp.int32),
        pltpu.VMEM((EP,), jnp.int32),
        pltpu.VMEM((HCH, D), jnp.float32),
        pltpu.VMEM((HCH, D), jnp.float32),
        pltpu.VMEM((HCH, D), jnp.float32),
        pltpu.VMEM((HCH, D), jnp.float32),
        pltpu.VMEM((D,), jnp.float32),
        pltpu.VMEM((EP,), jnp.float32),
        pltpu.VMEM((16, 16), jnp.float32),
        pltpu.SemaphoreType.DMA,
        pltpu.SemaphoreType.DMA,
        pltpu.SemaphoreType.DMA,
    ],
)
def _sc_head(a_hbm, b_hbm, w2_hbm, src_hbm, dst_hbm, s_hbm,
             sidx, didx, a0, a1, b0, b1, w2v, sbuf, ptile, sem0, sem1, semi):
    cid = lax.axis_index("c")
    sid = lax.axis_index("s")
    wid = cid * NS + sid
    ebase = wid * EP

    pltpu.make_async_copy(src_hbm.at[pl.ds(ebase, EP)], sidx, semi).start()
    pltpu.make_async_copy(dst_hbm.at[pl.ds(ebase, EP)], didx, semi).start()
    pltpu.sync_copy(w2_hbm, w2v)
    w2s = [w2v[pl.ds(j * 16, 16)] for j in range(D // 16)]
    lane = lax.iota(jnp.int32, 16)
    pltpu.make_async_copy(src_hbm.at[pl.ds(ebase, EP)], sidx, semi).wait()
    pltpu.make_async_copy(dst_hbm.at[pl.ds(ebase, EP)], didx, semi).wait()

    def fire(i, abuf, bbuf, sem):
        pltpu.make_async_copy(a_hbm.at[sidx.at[pl.ds(i * HCH, HCH)]], abuf,
                              sem).start()
        pltpu.make_async_copy(b_hbm.at[didx.at[pl.ds(i * HCH, HCH)]], bbuf,
                              sem).start()

    def drain(abuf, bbuf, sem):
        pltpu.make_async_copy(a_hbm.at[sidx.at[pl.ds(0, HCH)]], abuf, sem).wait()
        pltpu.make_async_copy(b_hbm.at[didx.at[pl.ds(0, HCH)]], bbuf, sem).wait()

    def compute(base, arows, brows, ngroups):
        def grp(g, cc):
            for rr in range(16):
                r = g * 16 + rr
                dot = jnp.zeros((16,), jnp.float32)
                for j in range(D // 16):
                    t = arows[r, pl.ds(j * 16, 16)] + brows[r, pl.ds(j * 16, 16)]
                    dot = dot + jnp.maximum(t, 0.0) * w2s[j]
                ptile[rr, pl.ds(0, 16)] = dot
            svec = jnp.zeros((16,), jnp.float32)
            for k in range(16):
                svec = svec + plsc.load_gather(
                    ptile, [lane, jnp.full((16,), k, jnp.int32)])
            sbuf[pl.ds(base + g * 16, 16)] = svec
            return cc

        lax.fori_loop(0, ngroups, grp, 0)

    fire(0, a0, b0, sem0)

    def pair(g, c):
        i0 = 2 * g
        fire(i0 + 1, a1, b1, sem1)
        drain(a0, b0, sem0)
        compute(i0 * HCH, a0, b0, HCH // 16)
        fire(i0 + 2, a0, b0, sem0)
        drain(a1, b1, sem1)
        compute((i0 + 1) * HCH, a1, b1, HCH // 16)
        return c

    lax.fori_loop(0, (HNCH - 2) // 2, pair, 0)
    fire(HNCH - 1, a1, b1, sem1)
    drain(a0, b0, sem0)
    compute((HNCH - 2) * HCH, a0, b0, HCH // 16)
    pltpu.make_async_copy(a_hbm.at[sidx.at[pl.ds(HNCH * HCH, HTAIL)]],
                          a0.at[pl.ds(0, HTAIL)], sem0).start()
    pltpu.make_async_copy(b_hbm.at[didx.at[pl.ds(HNCH * HCH, HTAIL)]],
                          b0.at[pl.ds(0, HTAIL)], sem0).start()
    drain(a1, b1, sem1)
    compute((HNCH - 1) * HCH, a1, b1, HCH // 16)
    pltpu.make_async_copy(a_hbm.at[sidx.at[pl.ds(0, HTAIL)]],
                          a0.at[pl.ds(0, HTAIL)], sem0).wait()
    pltpu.make_async_copy(b_hbm.at[didx.at[pl.ds(0, HTAIL)]],
                          b0.at[pl.ds(0, HTAIL)], sem0).wait()
    compute(HNCH * HCH, a0, b0, HTAIL // 16)
    pltpu.sync_copy(sbuf, s_hbm.at[pl.ds(wid * EP, EP)])


NB = 1000
GRID = N // NB


def _mlp_body(x_ref, a0_ref, a1_ref, w1_ref, b1_ref, g_ref, be_ref,
              w2_ref, b2_ref, o_ref):
    h = x_ref[...] + a0_ref[...] + a1_ref[...]
    t = jnp.dot(h, w1_ref[...], preferred_element_type=jnp.float32) + b1_ref[...]
    t = t * (g_ref[...] * BN_INV) + be_ref[...]
    t = jnp.maximum(t, 0.0)
    t = jnp.dot(t, w2_ref[...], preferred_element_type=jnp.float32) + b2_ref[...]
    o_ref[...] = jnp.maximum(t, 0.0)


def _mlp2_body(x_ref, a0_ref, a1_ref, w1_ref, b1_ref, g_ref, be_ref,
               w2_ref, b2_ref, mw1_ref, mb1_ref, a_out, b_out):
    h = x_ref[...] + a0_ref[...] + a1_ref[...]
    t = jnp.dot(h, w1_ref[...], preferred_element_type=jnp.float32) + b1_ref[...]
    t = t * (g_ref[...] * BN_INV) + be_ref[...]
    t = jnp.maximum(t, 0.0)
    t = jnp.dot(t, w2_ref[...], preferred_element_type=jnp.float32) + b2_ref[...]
    h2 = jnp.maximum(t, 0.0)
    a_out[...] = jnp.dot(h2, mw1_ref[0:H, :],
                         preferred_element_type=jnp.float32) + mb1_ref[...]
    b_out[...] = jnp.dot(h2, mw1_ref[H:2 * H, :],
                         preferred_element_type=jnp.float32)


_row_spec = pl.BlockSpec((NB, D), lambda i: (i, 0))
_row2_spec = pl.BlockSpec((NB, D), lambda i: (i + GRID, 0))
_w_spec = pl.BlockSpec((D, D), lambda i: (0, 0))
_b_spec = pl.BlockSpec((1, D), lambda i: (0, 0))

_mlp = pl.pallas_call(
    _mlp_body,
    grid=(GRID,),
    in_specs=[_row_spec, _row_spec, _row2_spec,
              _w_spec, _b_spec, _b_spec, _b_spec, _w_spec, _b_spec],
    out_specs=_row_spec,
    out_shape=jax.ShapeDtypeStruct((N, D), jnp.float32),
)

_mlp2 = pl.pallas_call(
    _mlp2_body,
    grid=(GRID,),
    in_specs=[_row_spec, _row_spec, _row2_spec,
              _w_spec, _b_spec, _b_spec, _b_spec, _w_spec, _b_spec,
              pl.BlockSpec((2 * H, D), lambda i: (0, 0)), _b_spec],
    out_specs=[_row_spec, _row_spec],
    out_shape=[jax.ShapeDtypeStruct((N, D), jnp.float32),
               jax.ShapeDtypeStruct((N, D), jnp.float32)],
)


def _softmax_body(s_ref, o_ref):
    s = s_ref[...]
    m = jnp.max(s)
    e = jnp.exp(s - m)
    o_ref[...] = e / jnp.sum(e)


_softmax = pl.pallas_call(
    _softmax_body,
    out_shape=jax.ShapeDtypeStruct((E // D, D), jnp.float32),
)


def kernel(x, edge_index, c0_w1, c0_b1, c0_g, c0_be, c0_w2, c0_b2,
           c1_w1, c1_b1, c1_g, c1_be, c1_w2, c1_b2,
           m_w1, m_b1, m_w2, m_b2):
    src = edge_index[0]
    dst = edge_index[1]
    r = lambda v: v.reshape(1, -1)

    agg0 = _sc_segsum(x, src, dst)
    h1 = _mlp(x, agg0, agg0, c0_w1, r(c0_b1), r(c0_g), r(c0_be),
              c0_w2, r(c0_b2))
    agg1 = _sc_segsum(h1, src, dst)
    a, b = _mlp2(h1, agg1, agg1, c1_w1, r(c1_b1), r(c1_g), r(c1_be),
                 c1_w2, r(c1_b2), m_w1, r(m_b1))
    s = _sc_head(a, b, m_w2.reshape(-1), src, dst)
    return _softmax(s.reshape(E // D, D)).reshape(-1)
